# baseline (speedup 1.0000x reference)
import functools

import jax
import jax.numpy as jnp
from jax.experimental import pallas as pl


def _gat(src, dst, es, ed, ws, wd, a_s, a_d, f1, f2):
    H, d = a_s.shape
    n_dst = dst.shape[0]
    hs = (src @ ws).reshape(src.shape[0], H, d)
    hd = (dst @ wd).reshape(n_dst, H, d)
    ls = jnp.sum(hs * a_s[None, :, :], axis=-1)
    ld = jnp.sum(hd * a_d[None, :, :], axis=-1)
    logits = jax.nn.leaky_relu(ls[es] + ld[ed], 0.2)
    m = jax.ops.segment_max(logits, ed, num_segments=n_dst)
    ex = jnp.exp(logits - m[ed])
    den = jax.ops.segment_sum(ex, ed, num_segments=n_dst)
    alpha = ex / (den[ed] + 1e-9)
    agg = jax.ops.segment_sum(hs[es] * alpha[:, :, None], ed, num_segments=n_dst)
    h = jax.nn.elu(agg.reshape(n_dst, H * d))
    return h + jax.nn.relu(h @ f1) @ f2


def _proj_body(x_ref, w_ref, b_ref, o_ref):
    o_ref[...] = x_ref[...] @ w_ref[...] + b_ref[...]


def _proj(x, w, b):
    n = x.shape[0]
    bn = 1024
    grid = (pl.cdiv(n, bn),)
    return pl.pallas_call(
        _proj_body,
        grid=grid,
        in_specs=[
            pl.BlockSpec((bn, x.shape[1]), lambda i: (i, 0)),
            pl.BlockSpec((x.shape[1], w.shape[1]), lambda i: (0, 0)),
            pl.BlockSpec((1, w.shape[1]), lambda i: (0, 0)),
        ],
        out_specs=pl.BlockSpec((bn, w.shape[1]), lambda i: (i, 0)),
        out_shape=jax.ShapeDtypeStruct((n, w.shape[1]), x.dtype),
    )(x, w, b.reshape(1, -1))


def kernel(wid, sent_words, asembed, edge_ws_src, edge_ws_dst, edge_sa_src,
           edge_sa_dst, etf, embed, fc_w, fc_b, asproj_w, proj_w, proj_b,
           tf_w, tf_b, pos_table, s2a_ws, s2a_wd, s2a_as, s2a_ad, s2a_f1,
           s2a_f2, a2s_ws, a2s_wd, a2s_as, a2s_ad, a2s_f1, a2s_f2, s2w_ws,
           s2w_wd, s2w_as, s2w_ad, s2w_f1, s2w_f2):
    sent = sent_words @ fc_w + fc_b
    asp = asembed @ asproj_w
    for _ in range(2):
        sent = _gat(asp, sent, edge_sa_dst, edge_sa_src,
                    a2s_ws, a2s_wd, a2s_as, a2s_ad, a2s_f1, a2s_f2)
        asp = _gat(sent, asp, edge_sa_src, edge_sa_dst,
                   s2a_ws, s2a_wd, s2a_as, s2a_ad, s2a_f1, s2a_f2)
    return _proj(jnp.concatenate([sent, asp], axis=0), proj_w, proj_b)


# speedup vs baseline: 31.2084x; 31.2084x over previous
import functools

import jax
import jax.numpy as jnp
from jax import lax
from jax.experimental import pallas as pl
from jax.experimental.pallas import tpu as pltpu
from jax.experimental.pallas import tpu_sc as plsc

F32 = jnp.float32
NS_PAD = 10240
NA_PAD = 128
E_PAD = 81920
NW = 32
K = 128
PER_W = E_PAD // NW
N_CHUNKS = PER_W // K


def _mm_body(x_ref, w_ref, b_ref, o_ref):
    o_ref[...] = x_ref[...] @ w_ref[...] + b_ref[...]


def _mm(x, w, b, block_rows=None):
    n, kin = x.shape
    ko = w.shape[1]
    if block_rows is None:
        block_rows = n
    grid = (n // block_rows,)
    return pl.pallas_call(
        _mm_body,
        grid=grid,
        in_specs=[
            pl.BlockSpec((block_rows, kin), lambda i: (i, 0)),
            pl.BlockSpec((kin, ko), lambda i: (0, 0)),
            pl.BlockSpec((1, ko), lambda i: (0, 0)),
        ],
        out_specs=pl.BlockSpec((block_rows, ko), lambda i: (i, 0)),
        out_shape=jax.ShapeDtypeStruct((n, ko), F32),
    )(x, w, b.reshape(1, -1))


def _pre_body(xs_ref, xd_ref, whs_ref, wlsb_ref, wldb_ref,
              hs_ref, lsb_ref, ldb_ref, cb_ref):
    xs = xs_ref[...]
    xd = xd_ref[...]
    hs_ref[...] = xs @ whs_ref[...]
    lsb = xs @ wlsb_ref[...]
    ldb = xd @ wldb_ref[...]
    lsb_ref[...] = lsb
    ldb_ref[...] = ldb
    t = jnp.max(lsb, axis=0) + jnp.max(ldb, axis=0)
    cb_ref[...] = jnp.maximum(t, 0.2 * t)[None, :]


def _pre(xs, xd, whs, wlsb, wldb):
    ns, nd = xs.shape[0], xd.shape[0]
    return pl.pallas_call(
        _pre_body,
        out_shape=[
            jax.ShapeDtypeStruct((ns, 64), F32),
            jax.ShapeDtypeStruct((ns, 64), F32),
            jax.ShapeDtypeStruct((nd, 64), F32),
            jax.ShapeDtypeStruct((1, 64), F32),
        ],
    )(xs, xd, whs, wlsb, wldb)


def _post_body(num_ref, den_ref, f1_ref, f2_ref, o_ref):
    num = jnp.sum(num_ref[...], axis=0)
    den = jnp.sum(den_ref[...], axis=0)
    agg = num / (den + 1e-30)
    h = jnp.where(agg > 0, agg, jnp.exp(jnp.minimum(agg, 0.0)) - 1.0)
    o_ref[...] = h + jnp.maximum(h @ f1_ref[...], 0.0) @ f2_ref[...]


def _post(num_p, den_p, f1, f2):
    n = num_p.shape[1]
    return pl.pallas_call(
        _post_body,
        out_shape=jax.ShapeDtypeStruct((n, 64), F32),
    )(num_p, den_p, f1, f2)


def _sc_shared(n_rows):
    stripe = n_rows // 16
    mesh = plsc.VectorSubcoreMesh(core_axis_name="c", subcore_axis_name="s")

    @functools.partial(
        pl.kernel,
        out_type=[jax.ShapeDtypeStruct((2 * n_rows, 64), F32),
                  jax.ShapeDtypeStruct((2 * n_rows, 64), F32)],
        mesh=mesh,
        compiler_params=pltpu.CompilerParams(use_tc_tiling_on_sc=False,
                                             needs_layout_passes=False),
        scratch_types=[
            pltpu.VMEM((K,), jnp.int32),
            pltpu.VMEM((K,), jnp.int32),
            pltpu.VMEM((K, 64), F32),
            pltpu.VMEM((K, 64), F32),
            pltpu.VMEM((K, 64), F32),
            pltpu.VMEM((K, 64), F32),
            pltpu.VMEM((K, 64), F32),
            pltpu.VMEM((1, 64), F32),
            pltpu.VMEM_SHARED((n_rows, 64), F32),
            pltpu.VMEM_SHARED((n_rows, 64), F32),
            pltpu.SemaphoreType.DMA,
            pltpu.SemaphoreType.DMA,
            pltpu.SemaphoreType.DMA,
        ],
    )
    def k(es_hbm, ed_hbm, lsb_hbm, ldb_hbm, hs_hbm, cb_hbm, z_hbm,
          num_out, den_out,
          es_v, ed_v, a_v, b_v, h_v, ex_v, r_v, cb_v,
          num_sp, den_sp, sem1, sem2, sem3):
        cid = lax.axis_index("c")
        sid = lax.axis_index("s")
        wid = sid * 2 + cid
        r0 = sid * stripe
        pltpu.sync_copy(z_hbm, num_sp.at[pl.ds(r0, stripe)])
        pltpu.sync_copy(z_hbm, den_sp.at[pl.ds(r0, stripe)])
        pltpu.sync_copy(cb_hbm, cb_v)
        plsc.subcore_barrier()
        cbs = [cb_v[0, pl.ds(16 * j, 16)] for j in range(4)]

        def chunk_body(c, _):
            b = wid * PER_W + c * K
            pltpu.sync_copy(es_hbm.at[pl.ds(b, K)], es_v)
            pltpu.sync_copy(ed_hbm.at[pl.ds(b, K)], ed_v)
            g1 = pltpu.async_copy(lsb_hbm.at[es_v], a_v, sem1)
            g2 = pltpu.async_copy(ldb_hbm.at[ed_v], b_v, sem2)
            g3 = pltpu.async_copy(hs_hbm.at[es_v], h_v, sem3)
            g1.wait()
            g2.wait()
            g3.wait()

            def edge_body(e, _):
                for j in range(4):
                    sl = pl.ds(16 * j, 16)
                    s = a_v[e, sl] + b_v[e, sl]
                    ex = jnp.exp(jnp.maximum(s, 0.2 * s) - cbs[j])
                    ex_v[e, sl] = ex
                    r_v[e, sl] = ex * h_v[e, sl]
                return 0

            lax.fori_loop(0, K, edge_body, 0)
            pltpu.sync_copy(ex_v, den_sp.at[ed_v], add=True)
            pltpu.sync_copy(r_v, num_sp.at[ed_v], add=True)
            return 0

        lax.fori_loop(0, N_CHUNKS, chunk_body, 0)
        plsc.subcore_barrier()
        o0 = cid * n_rows + r0
        pltpu.sync_copy(num_sp.at[pl.ds(r0, stripe)],
                        num_out.at[pl.ds(o0, stripe)])
        pltpu.sync_copy(den_sp.at[pl.ds(r0, stripe)],
                        den_out.at[pl.ds(o0, stripe)])

    return k


def _sc_private(n_rows):
    flat = n_rows * 64
    mesh = plsc.VectorSubcoreMesh(core_axis_name="c", subcore_axis_name="s")

    @functools.partial(
        pl.kernel,
        out_type=[jax.ShapeDtypeStruct((NW * flat,), F32),
                  jax.ShapeDtypeStruct((NW * flat,), F32)],
        mesh=mesh,
        compiler_params=pltpu.CompilerParams(use_tc_tiling_on_sc=False,
                                             needs_layout_passes=False),
        scratch_types=[
            pltpu.VMEM((K,), jnp.int32),
            pltpu.VMEM((K,), jnp.int32),
            pltpu.VMEM((K, 64), F32),
            pltpu.VMEM((K, 64), F32),
            pltpu.VMEM((K, 64), F32),
            pltpu.VMEM((1, 64), F32),
            pltpu.VMEM((flat,), F32),
            pltpu.VMEM((flat,), F32),
            pltpu.SemaphoreType.DMA,
            pltpu.SemaphoreType.DMA,
            pltpu.SemaphoreType.DMA,
        ],
    )
    def k(es_hbm, ed_hbm, lsb_hbm, ldb_hbm, hs_hbm, cb_hbm,
          num_out, den_out,
          es_v, ed_v, a_v, b_v, h_v, cb_v, pnum, pden,
          sem1, sem2, sem3):
        cid = lax.axis_index("c")
        sid = lax.axis_index("s")
        wid = sid * 2 + cid
        zero16 = jnp.zeros((16,), F32)

        def zero_body(i, _):
            pnum[pl.ds(16 * i, 16)] = zero16
            pden[pl.ds(16 * i, 16)] = zero16
            return 0

        lax.fori_loop(0, flat // 16, zero_body, 0)
        pltpu.sync_copy(cb_hbm, cb_v)
        cbs = [cb_v[0, pl.ds(16 * j, 16)] for j in range(4)]
        iota16 = lax.iota(jnp.int32, 16)

        def chunk_body(c, _):
            b = wid * PER_W + c * K
            pltpu.sync_copy(es_hbm.at[pl.ds(b, K)], es_v)
            pltpu.sync_copy(ed_hbm.at[pl.ds(b, K)], ed_v)
            g1 = pltpu.async_copy(lsb_hbm.at[es_v], a_v, sem1)
            g2 = pltpu.async_copy(ldb_hbm.at[ed_v], b_v, sem2)
            g3 = pltpu.async_copy(hs_hbm.at[es_v], h_v, sem3)
            g1.wait()
            g2.wait()
            g3.wait()

            def edge_body(e, _):
                edsp = plsc.load_gather(ed_v, [jnp.full((16,), e, jnp.int32)])
                base_idx = edsp * 64 + iota16
                for j in range(4):
                    sl = pl.ds(16 * j, 16)
                    s = a_v[e, sl] + b_v[e, sl]
                    ex = jnp.exp(jnp.maximum(s, 0.2 * s) - cbs[j])
                    idx = base_idx + (16 * j)
                    plsc.addupdate_scatter(pden, [idx], ex)
                    plsc.addupdate_scatter(pnum, [idx], ex * h_v[e, sl])
                return 0

            lax.fori_loop(0, K, edge_body, 0)
            return 0

        lax.fori_loop(0, N_CHUNKS, chunk_body, 0)
        o0 = wid * flat
        pltpu.sync_copy(pnum, num_out.at[pl.ds(o0, flat)])
        pltpu.sync_copy(pden, den_out.at[pl.ds(o0, flat)])

    return k


_sc_sent = _sc_shared(NS_PAD)
_sc_asp = _sc_private(NA_PAD)


def _blockdiag(a):
    eye = jnp.eye(8, dtype=a.dtype)
    blk = a[:, :, None, None] * eye[:, None, :, None]
    return jnp.broadcast_to(blk, (8, 8, 8, 8)).reshape(64, 64)


def _pad_rows(x, n):
    return jnp.concatenate(
        [x, jnp.zeros((n - x.shape[0], x.shape[1]), x.dtype)], axis=0)


def _pad_edges(e, fill):
    return jnp.concatenate(
        [e.astype(jnp.int32),
         jnp.full((E_PAD - e.shape[0],), fill, jnp.int32)])


def kernel(wid, sent_words, asembed, edge_ws_src, edge_ws_dst, edge_sa_src,
           edge_sa_dst, etf, embed, fc_w, fc_b, asproj_w, proj_w, proj_b,
           tf_w, tf_b, pos_table, s2a_ws, s2a_wd, s2a_as, s2a_ad, s2a_f1,
           s2a_f2, a2s_ws, a2s_wd, a2s_as, a2s_ad, a2s_f1, a2s_f2, s2w_ws,
           s2w_wd, s2w_as, s2w_ad, s2w_f1, s2w_f2):
    n_s, n_a = sent_words.shape[0], asembed.shape[0]
    sw = _pad_rows(sent_words, NS_PAD)
    ap = _pad_rows(asembed, NA_PAD)
    es_a = _pad_edges(edge_sa_dst, 0)
    ed_s = _pad_edges(edge_sa_src, n_s)
    es_s = _pad_edges(edge_sa_src, 0)
    ed_a = _pad_edges(edge_sa_dst, n_a)
    z = jnp.zeros((NS_PAD // 16, 64), F32)

    a2s_wlsb = a2s_ws @ _blockdiag(a2s_as)
    a2s_wldb = a2s_wd @ _blockdiag(a2s_ad)
    s2a_wlsb = s2a_ws @ _blockdiag(s2a_as)
    s2a_wldb = s2a_wd @ _blockdiag(s2a_ad)

    sent = _mm(sw, fc_w, fc_b, block_rows=1024)
    asp = _mm(ap, asproj_w, jnp.zeros((64,), F32))

    for _ in range(2):
        hs, lsb, ldb, cb = _pre(asp, sent, a2s_ws, a2s_wlsb, a2s_wldb)
        nump, denp = _sc_sent(es_a, ed_s, lsb, ldb, hs, cb, z)
        sent = _post(nump.reshape(2, NS_PAD, 64), denp.reshape(2, NS_PAD, 64),
                     a2s_f1, a2s_f2)
        hs, lsb, ldb, cb = _pre(sent, asp, s2a_ws, s2a_wlsb, s2a_wldb)
        nump, denp = _sc_asp(es_s, ed_a, lsb, ldb, hs, cb)
        asp = _post(nump.reshape(NW, NA_PAD, 64), denp.reshape(NW, NA_PAD, 64),
                    s2a_f1, s2a_f2)

    out_s = _mm(sent, proj_w, proj_b)
    out_a = _mm(asp, proj_w, proj_b)
    return jnp.concatenate([out_s[:n_s], out_a[:n_a]], axis=0)


# speedup vs baseline: 32.5382x; 1.0426x over previous
import functools

import jax
import jax.numpy as jnp
from jax import lax
from jax.experimental import pallas as pl
from jax.experimental.pallas import tpu as pltpu
from jax.experimental.pallas import tpu_sc as plsc

F32 = jnp.float32
NS_PAD = 10240
NACC = 10016
NA_PAD = 128
E_PAD = 81920
NW = 32
K = 128
PER_W = E_PAD // NW
N_CHUNKS = PER_W // K


def _mm_body(x_ref, w_ref, b_ref, o_ref):
    o_ref[...] = x_ref[...] @ w_ref[...] + b_ref[...]


def _mm(x, w, b, block_rows=None):
    n, kin = x.shape
    ko = w.shape[1]
    if block_rows is None:
        block_rows = n
    grid = (n // block_rows,)
    return pl.pallas_call(
        _mm_body,
        grid=grid,
        in_specs=[
            pl.BlockSpec((block_rows, kin), lambda i: (i, 0)),
            pl.BlockSpec((kin, ko), lambda i: (0, 0)),
            pl.BlockSpec((1, ko), lambda i: (0, 0)),
        ],
        out_specs=pl.BlockSpec((block_rows, ko), lambda i: (i, 0)),
        out_shape=jax.ShapeDtypeStruct((n, ko), F32),
    )(x, w, b.reshape(1, -1))


def _pre_body(xs_ref, xd_ref, whs_ref, wlsb_ref, wldb_ref,
              hs_ref, lsb_ref, ldb_ref, cb_ref):
    xs = xs_ref[...]
    xd = xd_ref[...]
    hs_ref[...] = xs @ whs_ref[...]
    lsb = xs @ wlsb_ref[...]
    ldb = xd @ wldb_ref[...]
    lsb_ref[...] = lsb
    ldb_ref[...] = ldb
    t = jnp.max(lsb, axis=0) + jnp.max(ldb, axis=0)
    cb_ref[...] = jnp.maximum(t, 0.2 * t)[None, :]


def _pre(xs, xd, whs, wlsb, wldb):
    ns, nd = xs.shape[0], xd.shape[0]
    return pl.pallas_call(
        _pre_body,
        out_shape=[
            jax.ShapeDtypeStruct((ns, 64), F32),
            jax.ShapeDtypeStruct((ns, 64), F32),
            jax.ShapeDtypeStruct((nd, 64), F32),
            jax.ShapeDtypeStruct((1, 64), F32),
        ],
    )(xs, xd, whs, wlsb, wldb)


def _ffn(h, f1_ref, f2_ref):
    h = jnp.where(h > 0, h, jnp.exp(jnp.minimum(h, 0.0)) - 1.0)
    return h + jnp.maximum(h @ f1_ref[...], 0.0) @ f2_ref[...]


def _post_sent_body(num_ref, den_ref, f1_ref, f2_ref, o_ref):
    num = jnp.sum(num_ref[...], axis=0)
    den16 = jnp.sum(den_ref[...], axis=0)
    n = num.shape[0]
    den = jnp.broadcast_to(den16[:, :, None], (n, 16, 4)).reshape(n, 64)
    o_ref[...] = _ffn(num / (den + 1e-30), f1_ref, f2_ref)


def _post_sent(num_p, den_p, f1, f2):
    k, n, _ = num_p.shape
    nb = 4
    bn = n // nb
    return pl.pallas_call(
        _post_sent_body,
        grid=(nb,),
        in_specs=[
            pl.BlockSpec((k, bn, 64), lambda i: (0, i, 0)),
            pl.BlockSpec((k, bn, 16), lambda i: (0, i, 0)),
            pl.BlockSpec(f1.shape, lambda i: (0, 0)),
            pl.BlockSpec(f2.shape, lambda i: (0, 0)),
        ],
        out_specs=pl.BlockSpec((bn, 64), lambda i: (i, 0)),
        out_shape=jax.ShapeDtypeStruct((n, 64), F32),
    )(num_p, den_p, f1, f2)


def _post_asp_body(acc_ref, f1_ref, f2_ref, o_ref):
    acc = jnp.sum(acc_ref[...], axis=0)
    den = acc[:, :64]
    num = acc[:, 64:]
    o_ref[...] = _ffn(num / (den + 1e-30), f1_ref, f2_ref)


def _post_asp(acc_p, f1, f2):
    n = acc_p.shape[1]
    return pl.pallas_call(
        _post_asp_body,
        out_shape=jax.ShapeDtypeStruct((n, 64), F32),
    )(acc_p, f1, f2)


def _sc_shared(n_rows):
    stripe = n_rows // 16
    mesh = plsc.VectorSubcoreMesh(core_axis_name="c", subcore_axis_name="s")

    @functools.partial(
        pl.kernel,
        out_type=[jax.ShapeDtypeStruct((2 * n_rows, 64), F32),
                  jax.ShapeDtypeStruct((2 * n_rows, 16), F32)],
        mesh=mesh,
        compiler_params=pltpu.CompilerParams(use_tc_tiling_on_sc=False,
                                             needs_layout_passes=False),
        scratch_types=[
            pltpu.VMEM((N_CHUNKS, K), jnp.int32),
            pltpu.VMEM((N_CHUNKS, K), jnp.int32),
            [pltpu.VMEM((K, 64), F32)] * 2,
            [pltpu.VMEM((K, 64), F32)] * 2,
            [pltpu.VMEM((K, 64), F32)] * 2,
            [pltpu.VMEM((K, 64), F32)] * 2,
            [pltpu.VMEM((K, 16), F32)] * 2,
            pltpu.VMEM((1, 64), F32),
            pltpu.VMEM_SHARED((n_rows, 64), F32),
            pltpu.VMEM_SHARED((n_rows, 16), F32),
            [pltpu.SemaphoreType.DMA] * 6,
        ],
    )
    def k(es_hbm, ed_hbm, lsb_hbm, ldb_hbm, hs_hbm, cb_hbm,
          num_out, den_out,
          es_st, ed_st, a_v, b_v, h_v, r_v, ex2_v, cb_v,
          num_sp, den_sp, gsem):
        cid = lax.axis_index("c")
        sid = lax.axis_index("s")
        wid = sid * 2 + cid
        r0 = sid * stripe
        pltpu.sync_copy(es_hbm.at[wid], es_st)
        pltpu.sync_copy(ed_hbm.at[wid], ed_st)
        pltpu.sync_copy(cb_hbm, cb_v)
        iota16 = lax.iota(jnp.int32, 16)
        m2 = (iota16 % 8) < 2
        dcols = [4 * j + (iota16 // 8) * 2 + (iota16 % 8) for j in range(4)]
        zero16 = jnp.zeros((16,), F32)

        def zero_body(i, _):
            for q in range(4):
                r_v[0][i, pl.ds(16 * q, 16)] = zero16
            ex2_v[0][i, :] = zero16
            return 0

        lax.fori_loop(0, K, zero_body, 0)
        nfull = stripe // K
        rem = stripe - nfull * K
        for t in range(nfull):
            pltpu.sync_copy(r_v[0], num_sp.at[pl.ds(r0 + t * K, K)])
            pltpu.sync_copy(ex2_v[0], den_sp.at[pl.ds(r0 + t * K, K)])
        if rem:
            pltpu.sync_copy(r_v[0].at[pl.ds(0, rem)],
                            num_sp.at[pl.ds(r0 + nfull * K, rem)])
            pltpu.sync_copy(ex2_v[0].at[pl.ds(0, rem)],
                            den_sp.at[pl.ds(r0 + nfull * K, rem)])
        plsc.subcore_barrier()
        cbs = [cb_v[0, pl.ds(16 * j, 16)] for j in range(4)]

        def gathers(c, p):
            return (
                pltpu.async_copy(lsb_hbm.at[es_st.at[c]], a_v[p], gsem[3 * p]),
                pltpu.async_copy(ldb_hbm.at[ed_st.at[c]], b_v[p], gsem[3 * p + 1]),
                pltpu.async_copy(hs_hbm.at[es_st.at[c]], h_v[p], gsem[3 * p + 2]),
            )

        def gwait(c, p):
            pltpu.make_async_copy(lsb_hbm.at[es_st.at[c]], a_v[p],
                                  gsem[3 * p]).wait()
            pltpu.make_async_copy(ldb_hbm.at[ed_st.at[c]], b_v[p],
                                  gsem[3 * p + 1]).wait()
            pltpu.make_async_copy(hs_hbm.at[es_st.at[c]], h_v[p],
                                  gsem[3 * p + 2]).wait()

        gathers(0, 0)

        def pair_body(c2, _):
            for p in range(2):
                c = c2 * 2 + p
                gwait(c, p)

                @pl.when(c + 1 < N_CHUNKS)
                def _():
                    gathers(c + 1, 1 - p)

                def edge_body(e, _):
                    ef = jnp.full((16,), e, jnp.int32)
                    for j in range(4):
                        sl = pl.ds(16 * j, 16)
                        s = a_v[p][e, sl] + b_v[p][e, sl]
                        ex = jnp.exp(jnp.maximum(s, 0.2 * s) - cbs[j])
                        r_v[p][e, sl] = ex * h_v[p][e, sl]
                        plsc.store_scatter(ex2_v[p], [ef, dcols[j]], ex,
                                           mask=m2)
                    return 0

                lax.fori_loop(0, K, edge_body, 0, unroll=4)
                pltpu.sync_copy(r_v[p], num_sp.at[ed_st.at[c]], add=True)
                pltpu.sync_copy(ex2_v[p], den_sp.at[ed_st.at[c]], add=True)
            return 0

        lax.fori_loop(0, N_CHUNKS // 2, pair_body, 0)
        plsc.subcore_barrier()
        o0 = cid * n_rows + r0
        pltpu.sync_copy(num_sp.at[pl.ds(r0, stripe)],
                        num_out.at[pl.ds(o0, stripe)])
        pltpu.sync_copy(den_sp.at[pl.ds(r0, stripe)],
                        den_out.at[pl.ds(o0, stripe)])

    return k


def _sc_private(n_rows):
    flat = n_rows * 128
    mesh = plsc.VectorSubcoreMesh(core_axis_name="c", subcore_axis_name="s")

    @functools.partial(
        pl.kernel,
        out_type=jax.ShapeDtypeStruct((NW * flat,), F32),
        mesh=mesh,
        compiler_params=pltpu.CompilerParams(use_tc_tiling_on_sc=False,
                                             needs_layout_passes=False),
        scratch_types=[
            pltpu.VMEM((N_CHUNKS, K), jnp.int32),
            pltpu.VMEM((N_CHUNKS, K), jnp.int32),
            [pltpu.VMEM((K, 64), F32)] * 2,
            [pltpu.VMEM((K, 64), F32)] * 2,
            [pltpu.VMEM((K, 64), F32)] * 2,
            pltpu.VMEM((1, 64), F32),
            pltpu.VMEM((flat,), F32),
            [pltpu.SemaphoreType.DMA] * 6,
        ],
    )
    def k(es_hbm, ed_hbm, lsb_hbm, ldb_hbm, hs_hbm, cb_hbm,
          acc_out,
          es_st, ed_st, a_v, b_v, h_v, cb_v, pacc, gsem):
        cid = lax.axis_index("c")
        sid = lax.axis_index("s")
        wid = sid * 2 + cid
        pltpu.sync_copy(es_hbm.at[wid], es_st)
        pltpu.sync_copy(ed_hbm.at[wid], ed_st)
        pltpu.sync_copy(cb_hbm, cb_v)
        zero16 = jnp.zeros((16,), F32)

        def zero_body(i, _):
            pacc[pl.ds(16 * i, 16)] = zero16
            return 0

        lax.fori_loop(0, flat // 16, zero_body, 0)
        cbs = [cb_v[0, pl.ds(16 * j, 16)] for j in range(4)]
        iota16 = lax.iota(jnp.int32, 16)

        def gathers(c, p):
            return (
                pltpu.async_copy(lsb_hbm.at[es_st.at[c]], a_v[p], gsem[3 * p]),
                pltpu.async_copy(ldb_hbm.at[ed_st.at[c]], b_v[p], gsem[3 * p + 1]),
                pltpu.async_copy(hs_hbm.at[es_st.at[c]], h_v[p], gsem[3 * p + 2]),
            )

        def gwait(c, p):
            pltpu.make_async_copy(lsb_hbm.at[es_st.at[c]], a_v[p],
                                  gsem[3 * p]).wait()
            pltpu.make_async_copy(ldb_hbm.at[ed_st.at[c]], b_v[p],
                                  gsem[3 * p + 1]).wait()
            pltpu.make_async_copy(hs_hbm.at[es_st.at[c]], h_v[p],
                                  gsem[3 * p + 2]).wait()

        gathers(0, 0)

        def pair_body(c2, _):
            for p in range(2):
                c = c2 * 2 + p
                gwait(c, p)

                @pl.when(c + 1 < N_CHUNKS)
                def _():
                    gathers(c + 1, 1 - p)

                def edge_body(e, _):
                    edsp = plsc.load_gather(
                        ed_st, [jnp.full((16,), c, jnp.int32),
                                jnp.full((16,), e, jnp.int32)])
                    base_idx = edsp * 128 + iota16
                    for j in range(4):
                        sl = pl.ds(16 * j, 16)
                        s = a_v[p][e, sl] + b_v[p][e, sl]
                        ex = jnp.exp(jnp.maximum(s, 0.2 * s) - cbs[j])
                        idx = base_idx + (16 * j)
                        plsc.addupdate_scatter(pacc, [idx], ex)
                        plsc.addupdate_scatter(pacc, [idx + 64],
                                               ex * h_v[p][e, sl])
                    return 0

                lax.fori_loop(0, K, edge_body, 0, unroll=2)
            return 0

        lax.fori_loop(0, N_CHUNKS // 2, pair_body, 0)
        pltpu.sync_copy(pacc, acc_out.at[pl.ds(wid * flat, flat)])

    return k


_sc_sent = _sc_shared(NACC)
_sc_asp = _sc_private(NA_PAD)


def _blockdiag(a):
    eye = jnp.eye(8, dtype=a.dtype)
    blk = a[:, :, None, None] * eye[:, None, :, None]
    return jnp.broadcast_to(blk, (8, 8, 8, 8)).reshape(64, 64)


def _pad_rows(x, n):
    return jnp.concatenate(
        [x, jnp.zeros((n - x.shape[0], x.shape[1]), x.dtype)], axis=0)


def _pad_edges(e, fill):
    return jnp.concatenate(
        [e.astype(jnp.int32),
         jnp.full((E_PAD - e.shape[0],), fill, jnp.int32)]
    ).reshape(NW, N_CHUNKS, K)


def kernel(wid, sent_words, asembed, edge_ws_src, edge_ws_dst, edge_sa_src,
           edge_sa_dst, etf, embed, fc_w, fc_b, asproj_w, proj_w, proj_b,
           tf_w, tf_b, pos_table, s2a_ws, s2a_wd, s2a_as, s2a_ad, s2a_f1,
           s2a_f2, a2s_ws, a2s_wd, a2s_as, a2s_ad, a2s_f1, a2s_f2, s2w_ws,
           s2w_wd, s2w_as, s2w_ad, s2w_f1, s2w_f2):
    n_s, n_a = sent_words.shape[0], asembed.shape[0]
    sw = _pad_rows(sent_words, NS_PAD)
    ap = _pad_rows(asembed, NA_PAD)
    es_a = _pad_edges(edge_sa_dst, 0)
    ed_s = _pad_edges(edge_sa_src, n_s)
    es_s = _pad_edges(edge_sa_src, 0)
    ed_a = _pad_edges(edge_sa_dst, n_a)

    a2s_wlsb = a2s_ws @ _blockdiag(a2s_as)
    a2s_wldb = a2s_wd @ _blockdiag(a2s_ad)
    s2a_wlsb = s2a_ws @ _blockdiag(s2a_as)
    s2a_wldb = s2a_wd @ _blockdiag(s2a_ad)

    sent = _mm(sw, fc_w, fc_b, block_rows=1024)
    asp = _mm(ap, asproj_w, jnp.zeros((64,), F32))

    for _ in range(2):
        hs, lsb, ldb, cb = _pre(asp, sent, a2s_ws, a2s_wlsb, a2s_wldb)
        nump, denp = _sc_sent(es_a, ed_s, lsb, ldb, hs, cb)
        sent = _post_sent(nump.reshape(2, NACC, 64), denp.reshape(2, NACC, 16),
                          a2s_f1, a2s_f2)
        hs, lsb, ldb, cb = _pre(sent, asp, s2a_ws, s2a_wlsb, s2a_wldb)
        accp = _sc_asp(es_s, ed_a, lsb, ldb, hs, cb)
        asp = _post_asp(accp.reshape(NW, NA_PAD, 128), s2a_f1, s2a_f2)

    out_s = _mm(sent, proj_w, proj_b)
    out_a = _mm(asp, proj_w, proj_b)
    return jnp.concatenate([out_s[:n_s], out_a[:n_a]], axis=0)


# speedup vs baseline: 47.6651x; 1.4649x over previous
import functools

import jax
import jax.numpy as jnp
from jax import lax
from jax.experimental import pallas as pl
from jax.experimental.pallas import tpu as pltpu
from jax.experimental.pallas import tpu_sc as plsc

F32 = jnp.float32
NS_PAD = 10240
NACC = 10016
NA_PAD = 128
E_PAD = 81920
NW = 32
K = 128
PER_W = E_PAD // NW
N_CHUNKS = PER_W // K


def _mm_body(x_ref, w_ref, b_ref, o_ref):
    o_ref[...] = x_ref[...] @ w_ref[...] + b_ref[...]


def _mm(x, w, b, block_rows=None):
    n, kin = x.shape
    ko = w.shape[1]
    if block_rows is None:
        block_rows = n
    grid = (n // block_rows,)
    return pl.pallas_call(
        _mm_body,
        grid=grid,
        in_specs=[
            pl.BlockSpec((block_rows, kin), lambda i: (i, 0)),
            pl.BlockSpec((kin, ko), lambda i: (0, 0)),
            pl.BlockSpec((1, ko), lambda i: (0, 0)),
        ],
        out_specs=pl.BlockSpec((block_rows, ko), lambda i: (i, 0)),
        out_shape=jax.ShapeDtypeStruct((n, ko), F32),
    )(x, w, b.reshape(1, -1))


def _pre_body(xs_ref, xd_ref, whs_ref, wlsb_ref, wldb_ref,
              hs_ref, lsb_ref, ldb_ref, cb_ref):
    xs = xs_ref[...]
    xd = xd_ref[...]
    hs_ref[...] = xs @ whs_ref[...]
    lsb = xs @ wlsb_ref[...]
    ldb = xd @ wldb_ref[...]
    lsb_ref[...] = lsb
    ldb_ref[...] = ldb
    t = jnp.max(lsb, axis=0) + jnp.max(ldb, axis=0)
    cb_ref[...] = jnp.maximum(t, 0.2 * t)[None, :]


def _pre(xs, xd, whs, wlsb, wldb):
    ns, nd = xs.shape[0], xd.shape[0]
    return pl.pallas_call(
        _pre_body,
        out_shape=[
            jax.ShapeDtypeStruct((ns, 64), F32),
            jax.ShapeDtypeStruct((ns, 64), F32),
            jax.ShapeDtypeStruct((nd, 64), F32),
            jax.ShapeDtypeStruct((1, 64), F32),
        ],
    )(xs, xd, whs, wlsb, wldb)


def _ffn(h, f1_ref, f2_ref):
    h = jnp.where(h > 0, h, jnp.exp(jnp.minimum(h, 0.0)) - 1.0)
    return h + jnp.maximum(h @ f1_ref[...], 0.0) @ f2_ref[...]


def _post_sent_body(num_ref, den_ref, f1_ref, f2_ref, o_ref):
    num = jnp.sum(num_ref[...], axis=0)
    den16 = jnp.sum(den_ref[...], axis=0)
    n = num.shape[0]
    den = jnp.broadcast_to(den16[:, :, None], (n, 16, 4)).reshape(n, 64)
    o_ref[...] = _ffn(num / (den + 1e-30), f1_ref, f2_ref)


def _post_sent(num_p, den_p, f1, f2):
    k, n, _ = num_p.shape
    nb = 4
    bn = n // nb
    return pl.pallas_call(
        _post_sent_body,
        grid=(nb,),
        in_specs=[
            pl.BlockSpec((k, bn, 64), lambda i: (0, i, 0)),
            pl.BlockSpec((k, bn, 16), lambda i: (0, i, 0)),
            pl.BlockSpec(f1.shape, lambda i: (0, 0)),
            pl.BlockSpec(f2.shape, lambda i: (0, 0)),
        ],
        out_specs=pl.BlockSpec((bn, 64), lambda i: (i, 0)),
        out_shape=jax.ShapeDtypeStruct((n, 64), F32),
    )(num_p, den_p, f1, f2)


def _post_asp_body(acc_ref, f1_ref, f2_ref, o_ref):
    acc = jnp.sum(acc_ref[...], axis=0)
    den = acc[:, :64]
    num = acc[:, 64:]
    o_ref[...] = _ffn(num / (den + 1e-30), f1_ref, f2_ref)


def _post_asp(acc_p, f1, f2):
    n = acc_p.shape[1]
    return pl.pallas_call(
        _post_asp_body,
        out_shape=jax.ShapeDtypeStruct((n, 64), F32),
    )(acc_p, f1, f2)


def _sc_shared(n_rows):
    stripe = n_rows // 16
    mesh = plsc.VectorSubcoreMesh(core_axis_name="c", subcore_axis_name="s")

    @functools.partial(
        pl.kernel,
        out_type=[jax.ShapeDtypeStruct((2 * n_rows, 64), F32),
                  jax.ShapeDtypeStruct((2 * n_rows, 16), F32)],
        mesh=mesh,
        compiler_params=pltpu.CompilerParams(use_tc_tiling_on_sc=False,
                                             needs_layout_passes=False),
        scratch_types=[
            pltpu.VMEM((N_CHUNKS, K), jnp.int32),
            pltpu.VMEM((N_CHUNKS, K), jnp.int32),
            [pltpu.VMEM((K, 64), F32)] * 2,
            [pltpu.VMEM((K, 64), F32)] * 2,
            [pltpu.VMEM((K, 64), F32)] * 2,
            [pltpu.VMEM((K, 64), F32)] * 2,
            [pltpu.VMEM((K, 16), F32)] * 2,
            pltpu.VMEM((1, 64), F32),
            pltpu.VMEM_SHARED((n_rows, 64), F32),
            pltpu.VMEM_SHARED((n_rows, 16), F32),
            [pltpu.SemaphoreType.DMA] * 6,
        ],
    )
    def k(es_hbm, ed_hbm, lsb_hbm, ldb_hbm, hs_hbm, cb_hbm,
          num_out, den_out,
          es_st, ed_st, a_v, b_v, h_v, r_v, ex2_v, cb_v,
          num_sp, den_sp, gsem):
        cid = lax.axis_index("c")
        sid = lax.axis_index("s")
        wid = sid * 2 + cid
        r0 = sid * stripe
        pltpu.sync_copy(es_hbm.at[wid], es_st)
        pltpu.sync_copy(ed_hbm.at[wid], ed_st)
        pltpu.sync_copy(cb_hbm, cb_v)
        iota16 = lax.iota(jnp.int32, 16)
        m2 = (iota16 % 8) < 2
        dcols = [4 * j + (iota16 // 8) * 2 + (iota16 % 8) for j in range(4)]
        zero16 = jnp.zeros((16,), F32)

        def zero_body(i, _):
            for q in range(4):
                r_v[0][i, pl.ds(16 * q, 16)] = zero16
            ex2_v[0][i, :] = zero16
            return 0

        lax.fori_loop(0, K, zero_body, 0)
        nfull = stripe // K
        rem = stripe - nfull * K
        for t in range(nfull):
            pltpu.sync_copy(r_v[0], num_sp.at[pl.ds(r0 + t * K, K)])
            pltpu.sync_copy(ex2_v[0], den_sp.at[pl.ds(r0 + t * K, K)])
        if rem:
            pltpu.sync_copy(r_v[0].at[pl.ds(0, rem)],
                            num_sp.at[pl.ds(r0 + nfull * K, rem)])
            pltpu.sync_copy(ex2_v[0].at[pl.ds(0, rem)],
                            den_sp.at[pl.ds(r0 + nfull * K, rem)])
        plsc.subcore_barrier()
        cbs = [cb_v[0, pl.ds(16 * j, 16)] for j in range(4)]

        def gathers(c, p):
            return (
                pltpu.async_copy(lsb_hbm.at[es_st.at[c]], a_v[p], gsem[3 * p]),
                pltpu.async_copy(ldb_hbm.at[ed_st.at[c]], b_v[p], gsem[3 * p + 1]),
                pltpu.async_copy(hs_hbm.at[es_st.at[c]], h_v[p], gsem[3 * p + 2]),
            )

        def gwait(c, p):
            pltpu.make_async_copy(lsb_hbm.at[es_st.at[c]], a_v[p],
                                  gsem[3 * p]).wait()
            pltpu.make_async_copy(ldb_hbm.at[ed_st.at[c]], b_v[p],
                                  gsem[3 * p + 1]).wait()
            pltpu.make_async_copy(hs_hbm.at[es_st.at[c]], h_v[p],
                                  gsem[3 * p + 2]).wait()

        gathers(0, 0)

        def pair_body(c2, _):
            for p in range(2):
                c = c2 * 2 + p
                gwait(c, p)

                @pl.when(c + 1 < N_CHUNKS)
                def _():
                    gathers(c + 1, 1 - p)

                @plsc.parallel_loop(0, K, 1, unroll=4)
                def _(e):
                    ef = jnp.full((16,), e, jnp.int32)
                    for j in range(4):
                        sl = pl.ds(16 * j, 16)
                        s = a_v[p][e, sl] + b_v[p][e, sl]
                        ex = jnp.exp(jnp.maximum(s, 0.2 * s) - cbs[j])
                        r_v[p][e, sl] = ex * h_v[p][e, sl]
                        plsc.store_scatter(ex2_v[p], [ef, dcols[j]], ex,
                                           mask=m2)
                pltpu.sync_copy(r_v[p], num_sp.at[ed_st.at[c]], add=True)
                pltpu.sync_copy(ex2_v[p], den_sp.at[ed_st.at[c]], add=True)
            return 0

        lax.fori_loop(0, N_CHUNKS // 2, pair_body, 0)
        plsc.subcore_barrier()
        o0 = cid * n_rows + r0
        pltpu.sync_copy(num_sp.at[pl.ds(r0, stripe)],
                        num_out.at[pl.ds(o0, stripe)])
        pltpu.sync_copy(den_sp.at[pl.ds(r0, stripe)],
                        den_out.at[pl.ds(o0, stripe)])

    return k


def _sc_private(n_rows):
    flat = n_rows * 128
    mesh = plsc.VectorSubcoreMesh(core_axis_name="c", subcore_axis_name="s")

    @functools.partial(
        pl.kernel,
        out_type=jax.ShapeDtypeStruct((NW * flat,), F32),
        mesh=mesh,
        compiler_params=pltpu.CompilerParams(use_tc_tiling_on_sc=False,
                                             needs_layout_passes=False),
        scratch_types=[
            pltpu.VMEM((N_CHUNKS, K), jnp.int32),
            pltpu.VMEM((N_CHUNKS, K), jnp.int32),
            [pltpu.VMEM((K, 64), F32)] * 2,
            [pltpu.VMEM((K, 64), F32)] * 2,
            [pltpu.VMEM((K, 64), F32)] * 2,
            pltpu.VMEM((1, 64), F32),
            pltpu.VMEM((flat,), F32),
            [pltpu.SemaphoreType.DMA] * 6,
        ],
    )
    def k(es_hbm, ed_hbm, lsb_hbm, ldb_hbm, hs_hbm, cb_hbm,
          acc_out,
          es_st, ed_st, a_v, b_v, h_v, cb_v, pacc, gsem):
        cid = lax.axis_index("c")
        sid = lax.axis_index("s")
        wid = sid * 2 + cid
        pltpu.sync_copy(es_hbm.at[wid], es_st)
        pltpu.sync_copy(ed_hbm.at[wid], ed_st)
        pltpu.sync_copy(cb_hbm, cb_v)
        zero16 = jnp.zeros((16,), F32)

        def zero_body(i, _):
            pacc[pl.ds(16 * i, 16)] = zero16
            return 0

        lax.fori_loop(0, flat // 16, zero_body, 0)
        cbs = [cb_v[0, pl.ds(16 * j, 16)] for j in range(4)]
        iota16 = lax.iota(jnp.int32, 16)

        def gathers(c, p):
            return (
                pltpu.async_copy(lsb_hbm.at[es_st.at[c]], a_v[p], gsem[3 * p]),
                pltpu.async_copy(ldb_hbm.at[ed_st.at[c]], b_v[p], gsem[3 * p + 1]),
                pltpu.async_copy(hs_hbm.at[es_st.at[c]], h_v[p], gsem[3 * p + 2]),
            )

        def gwait(c, p):
            pltpu.make_async_copy(lsb_hbm.at[es_st.at[c]], a_v[p],
                                  gsem[3 * p]).wait()
            pltpu.make_async_copy(ldb_hbm.at[ed_st.at[c]], b_v[p],
                                  gsem[3 * p + 1]).wait()
            pltpu.make_async_copy(hs_hbm.at[es_st.at[c]], h_v[p],
                                  gsem[3 * p + 2]).wait()

        gathers(0, 0)

        def pair_body(c2, _):
            for p in range(2):
                c = c2 * 2 + p
                gwait(c, p)

                @pl.when(c + 1 < N_CHUNKS)
                def _():
                    gathers(c + 1, 1 - p)

                @plsc.parallel_loop(0, K, 1, unroll=4)
                def _(e):
                    edsp = plsc.load_gather(
                        ed_st, [jnp.full((16,), c, jnp.int32),
                                jnp.full((16,), e, jnp.int32)])
                    base_idx = edsp * 128 + iota16
                    for j in range(4):
                        sl = pl.ds(16 * j, 16)
                        s = a_v[p][e, sl] + b_v[p][e, sl]
                        ex = jnp.exp(jnp.maximum(s, 0.2 * s) - cbs[j])
                        idx = base_idx + (16 * j)
                        plsc.addupdate_scatter(pacc, [idx], ex)
                        plsc.addupdate_scatter(pacc, [idx + 64],
                                               ex * h_v[p][e, sl])
            return 0

        lax.fori_loop(0, N_CHUNKS // 2, pair_body, 0)
        pltpu.sync_copy(pacc, acc_out.at[pl.ds(wid * flat, flat)])

    return k


_sc_sent = _sc_shared(NACC)
_sc_asp = _sc_private(NA_PAD)


def _blockdiag(a):
    eye = jnp.eye(8, dtype=a.dtype)
    blk = a[:, :, None, None] * eye[:, None, :, None]
    return jnp.broadcast_to(blk, (8, 8, 8, 8)).reshape(64, 64)


def _pad_rows(x, n):
    return jnp.concatenate(
        [x, jnp.zeros((n - x.shape[0], x.shape[1]), x.dtype)], axis=0)


def _pad_edges(e, fill):
    return jnp.concatenate(
        [e.astype(jnp.int32),
         jnp.full((E_PAD - e.shape[0],), fill, jnp.int32)]
    ).reshape(NW, N_CHUNKS, K)


def kernel(wid, sent_words, asembed, edge_ws_src, edge_ws_dst, edge_sa_src,
           edge_sa_dst, etf, embed, fc_w, fc_b, asproj_w, proj_w, proj_b,
           tf_w, tf_b, pos_table, s2a_ws, s2a_wd, s2a_as, s2a_ad, s2a_f1,
           s2a_f2, a2s_ws, a2s_wd, a2s_as, a2s_ad, a2s_f1, a2s_f2, s2w_ws,
           s2w_wd, s2w_as, s2w_ad, s2w_f1, s2w_f2):
    n_s, n_a = sent_words.shape[0], asembed.shape[0]
    sw = _pad_rows(sent_words, NS_PAD)
    ap = _pad_rows(asembed, NA_PAD)
    es_a = _pad_edges(edge_sa_dst, 0)
    ed_s = _pad_edges(edge_sa_src, n_s)
    es_s = _pad_edges(edge_sa_src, 0)
    ed_a = _pad_edges(edge_sa_dst, n_a)

    a2s_wlsb = a2s_ws @ _blockdiag(a2s_as)
    a2s_wldb = a2s_wd @ _blockdiag(a2s_ad)
    s2a_wlsb = s2a_ws @ _blockdiag(s2a_as)
    s2a_wldb = s2a_wd @ _blockdiag(s2a_ad)

    sent = _mm(sw, fc_w, fc_b, block_rows=1024)
    asp = _mm(ap, asproj_w, jnp.zeros((64,), F32))

    for _ in range(2):
        hs, lsb, ldb, cb = _pre(asp, sent, a2s_ws, a2s_wlsb, a2s_wldb)
        nump, denp = _sc_sent(es_a, ed_s, lsb, ldb, hs, cb)
        sent = _post_sent(nump.reshape(2, NACC, 64), denp.reshape(2, NACC, 16),
                          a2s_f1, a2s_f2)
        hs, lsb, ldb, cb = _pre(sent, asp, s2a_ws, s2a_wlsb, s2a_wldb)
        accp = _sc_asp(es_s, ed_a, lsb, ldb, hs, cb)
        asp = _post_asp(accp.reshape(NW, NA_PAD, 128), s2a_f1, s2a_f2)

    out_s = _mm(sent, proj_w, proj_b)
    out_a = _mm(asp, proj_w, proj_b)
    return jnp.concatenate([out_s[:n_s], out_a[:n_a]], axis=0)
